# baseline (speedup 1.0000x reference)
import jax
import jax.numpy as jnp
from jax import lax
from jax.experimental import pallas as pl
from jax.experimental.pallas import tpu as pltpu
from jax.experimental.pallas import tpu_sc as plsc

SEQ_LEN = 200
BATCH = 4096
EMBEDDING_DIM = 32
NUM_IDX = SEQ_LEN * BATCH
NUM_WORKERS = 32
PER_WORKER = NUM_IDX // NUM_WORKERS
STREAM_W = 128
CHUNK = 1280
STREAM_ROWS = 1024
DMA_ROWS = CHUNK - STREAM_ROWS
NSTREAM = STREAM_ROWS // STREAM_W
NCHUNK = PER_WORKER // CHUNK
NBUF = 2


def _gather_rows(weight, idx_grp):
    mesh = plsc.VectorSubcoreMesh(core_axis_name="c", subcore_axis_name="s")

    @pl.kernel(
        out_type=jax.ShapeDtypeStruct(
            (NUM_WORKERS, NCHUNK, CHUNK, EMBEDDING_DIM), weight.dtype
        ),
        mesh=mesh,
        scratch_types=[
            pltpu.VMEM((NBUF, STREAM_ROWS), jnp.int32),
            pltpu.VMEM((NBUF, CHUNK, EMBEDDING_DIM), jnp.float32),
            pltpu.SMEM((NBUF, DMA_ROWS), jnp.int32),
            pltpu.VMEM_SHARED((16, NBUF, DMA_ROWS), jnp.int32),
        ]
        + [pltpu.SemaphoreType.DMA] * (2 * NBUF),
        compiler_params=pltpu.CompilerParams(use_tc_tiling_on_sc=False),
    )
    def gather_kernel(w_hbm, i_hbm, o_hbm, idx_v, rows_v, idx_s, idx_sp, *sems):
        sid = lax.axis_index("s")
        wid = sid * 2 + lax.axis_index("c")
        ssems = sems[:NBUF]
        dsems = sems[NBUF:]

        def load_and_fire(g, b):
            pltpu.sync_copy(i_hbm.at[wid, g, pl.ds(0, STREAM_ROWS)], idx_v.at[b])
            pltpu.sync_copy(
                i_hbm.at[wid, g, pl.ds(STREAM_ROWS, DMA_ROWS)], idx_sp.at[sid, b]
            )
            pltpu.sync_copy(idx_sp.at[sid, b], idx_s.at[b])
            for j in range(NSTREAM):
                pltpu.async_copy(
                    w_hbm.at[idx_v.at[b, pl.ds(j * STREAM_W, STREAM_W)]],
                    rows_v.at[b, pl.ds(j * STREAM_W, STREAM_W)],
                    ssems[b],
                )

            def issue_row(r, carry):
                i = idx_s[b, r]
                pltpu.async_copy(
                    w_hbm.at[i],
                    rows_v.at[b, STREAM_ROWS + r],
                    dsems[b],
                )
                return carry

            lax.fori_loop(0, DMA_ROWS, issue_row, 0)

        def drain(b):
            for j in range(NSTREAM):
                pltpu.make_async_copy(
                    w_hbm.at[idx_v.at[b, pl.ds(j * STREAM_W, STREAM_W)]],
                    rows_v.at[b, pl.ds(j * STREAM_W, STREAM_W)],
                    ssems[b],
                ).wait()
            pltpu.make_async_copy(
                w_hbm.at[pl.ds(0, DMA_ROWS)],
                rows_v.at[b, pl.ds(STREAM_ROWS, DMA_ROWS)],
                dsems[b],
            ).wait()

        for b in range(NBUF):
            load_and_fire(b, b)

        def ring_body(p, carry):
            for b in range(NBUF):
                g = NBUF * p + b
                drain(b)
                pltpu.sync_copy(rows_v.at[b], o_hbm.at[wid, g])

                @pl.when(g + NBUF < NCHUNK)
                def _():
                    load_and_fire(g + NBUF, b)

            return carry

        lax.fori_loop(0, NCHUNK // NBUF, ring_body, 0)

    return gather_kernel(weight, idx_grp)


def kernel(inputs, weight):
    idx_grp = inputs.reshape(NUM_WORKERS, NCHUNK, CHUNK)
    out = _gather_rows(weight, idx_grp)
    return out.reshape(SEQ_LEN, BATCH, EMBEDDING_DIM)


# speedup vs baseline: 1.0045x; 1.0045x over previous
import jax
import jax.numpy as jnp
from jax import lax
from jax.experimental import pallas as pl
from jax.experimental.pallas import tpu as pltpu
from jax.experimental.pallas import tpu_sc as plsc

SEQ_LEN = 200
BATCH = 4096
EMBEDDING_DIM = 32
NUM_IDX = SEQ_LEN * BATCH
NUM_WORKERS = 32
PER_WORKER = NUM_IDX // NUM_WORKERS
STREAM_W = 128
CHUNK = 640
NSTREAM = CHUNK // STREAM_W
NCHUNK = PER_WORKER // CHUNK
NBUF = 4


def _gather_rows(weight, idx_grp):
    mesh = plsc.VectorSubcoreMesh(core_axis_name="c", subcore_axis_name="s")

    @pl.kernel(
        out_type=jax.ShapeDtypeStruct(
            (NUM_WORKERS, NCHUNK, CHUNK, EMBEDDING_DIM), weight.dtype
        ),
        mesh=mesh,
        scratch_types=[
            pltpu.VMEM((NBUF, NSTREAM, STREAM_W), jnp.int32),
            pltpu.VMEM((NBUF, CHUNK, EMBEDDING_DIM), jnp.float32),
        ]
        + [pltpu.SemaphoreType.DMA] * (2 * NBUF),
        compiler_params=pltpu.CompilerParams(use_tc_tiling_on_sc=False),
    )
    def gather_kernel(w_hbm, i_hbm, o_hbm, idx_v, rows_v, *sems):
        wid = lax.axis_index("s") * 2 + lax.axis_index("c")
        gsems = sems[:NBUF]
        osems = sems[NBUF:]

        def load_and_fire(g, b):
            pltpu.sync_copy(i_hbm.at[wid, g], idx_v.at[b])
            for j in range(NSTREAM):
                pltpu.async_copy(
                    w_hbm.at[idx_v.at[b, j]],
                    rows_v.at[b, pl.ds(j * STREAM_W, STREAM_W)],
                    gsems[b],
                )

        def drain(b):
            for j in range(NSTREAM):
                pltpu.make_async_copy(
                    w_hbm.at[idx_v.at[b, j]],
                    rows_v.at[b, pl.ds(j * STREAM_W, STREAM_W)],
                    gsems[b],
                ).wait()

        for b in range(NBUF):
            load_and_fire(b, b)

        def ring_body(p, carry):
            for b in range(NBUF):
                g = NBUF * p + b
                drain(b)
                pltpu.async_copy(rows_v.at[b], o_hbm.at[wid, g], osems[b])

                @pl.when(g + NBUF < NCHUNK)
                def _():
                    pltpu.make_async_copy(
                        rows_v.at[b], o_hbm.at[wid, g], osems[b]
                    ).wait()
                    load_and_fire(g + NBUF, b)

            return carry

        lax.fori_loop(0, NCHUNK // NBUF, ring_body, 0)

        def tail_body(q, carry):
            for b in range(NBUF):
                pltpu.make_async_copy(
                    rows_v.at[b], o_hbm.at[wid, NCHUNK - NBUF + b], osems[b]
                ).wait()
            return carry

        lax.fori_loop(0, 1, tail_body, 0)

    return gather_kernel(weight, idx_grp)


def kernel(inputs, weight):
    idx_grp = inputs.reshape(NUM_WORKERS, NCHUNK, NSTREAM, STREAM_W)
    out = _gather_rows(weight, idx_grp)
    return out.reshape(SEQ_LEN, BATCH, EMBEDDING_DIM)


# speedup vs baseline: 1.0175x; 1.0129x over previous
import jax
import jax.numpy as jnp
from jax import lax
from jax.experimental import pallas as pl
from jax.experimental.pallas import tpu as pltpu
from jax.experimental.pallas import tpu_sc as plsc

SEQ_LEN = 200
BATCH = 4096
EMBEDDING_DIM = 32
NUM_IDX = SEQ_LEN * BATCH
NUM_WORKERS = 32
PER_WORKER = NUM_IDX // NUM_WORKERS
STREAM_W = 128
CHUNK = 640
NSTREAM = CHUNK // STREAM_W
NCHUNK = PER_WORKER // CHUNK
NBUF = 2


def _gather_rows(weight, idx_grp):
    mesh = plsc.VectorSubcoreMesh(core_axis_name="c", subcore_axis_name="s")

    @pl.kernel(
        out_type=jax.ShapeDtypeStruct(
            (NUM_WORKERS, NCHUNK, CHUNK, EMBEDDING_DIM), weight.dtype
        ),
        mesh=mesh,
        scratch_types=[
            pltpu.VMEM((NBUF, NSTREAM, STREAM_W), jnp.int32),
            pltpu.VMEM((NBUF, CHUNK, EMBEDDING_DIM), jnp.float32),
            pltpu.VMEM_SHARED((16, NBUF, CHUNK, EMBEDDING_DIM), jnp.float32),
        ]
        + [pltpu.SemaphoreType.DMA] * (2 * NBUF),
        compiler_params=pltpu.CompilerParams(use_tc_tiling_on_sc=False),
    )
    def gather_kernel(w_hbm, i_hbm, o_hbm, idx_v, rows_v, sp_rows, *sems):
        sid = lax.axis_index("s")
        wid = sid * 2 + lax.axis_index("c")
        gsems = sems[:NBUF]
        osems = sems[NBUF:]

        def load_and_fire(g, b):
            pltpu.sync_copy(i_hbm.at[wid, g], idx_v.at[b])
            for j in range(NSTREAM):
                pltpu.async_copy(
                    w_hbm.at[idx_v.at[b, j]],
                    rows_v.at[b, pl.ds(j * STREAM_W, STREAM_W)],
                    gsems[b],
                )

        def drain(b):
            for j in range(NSTREAM):
                pltpu.make_async_copy(
                    w_hbm.at[idx_v.at[b, j]],
                    rows_v.at[b, pl.ds(j * STREAM_W, STREAM_W)],
                    gsems[b],
                ).wait()

        for b in range(NBUF):
            load_and_fire(b, b)

        def ring_body(p, carry):
            for b in range(NBUF):
                g = NBUF * p + b
                drain(b)

                @pl.when(g >= NBUF)
                def _():
                    pltpu.make_async_copy(
                        sp_rows.at[sid, b], o_hbm.at[wid, g - NBUF], osems[b]
                    ).wait()

                pltpu.sync_copy(rows_v.at[b], sp_rows.at[sid, b])
                pltpu.async_copy(sp_rows.at[sid, b], o_hbm.at[wid, g], osems[b])

                @pl.when(g + NBUF < NCHUNK)
                def _():
                    load_and_fire(g + NBUF, b)

            return carry

        lax.fori_loop(0, NCHUNK // NBUF, ring_body, 0)

        for b in range(NBUF):
            pltpu.make_async_copy(
                sp_rows.at[sid, b], o_hbm.at[wid, NCHUNK - NBUF + b], osems[b]
            ).wait()

    return gather_kernel(weight, idx_grp)


def kernel(inputs, weight):
    idx_grp = inputs.reshape(NUM_WORKERS, NCHUNK, NSTREAM, STREAM_W)
    out = _gather_rows(weight, idx_grp)
    return out.reshape(SEQ_LEN, BATCH, EMBEDDING_DIM)


# speedup vs baseline: 1.0187x; 1.0012x over previous
import jax
import jax.numpy as jnp
from jax import lax
from jax.experimental import pallas as pl
from jax.experimental.pallas import tpu as pltpu
from jax.experimental.pallas import tpu_sc as plsc

SEQ_LEN = 200
BATCH = 4096
EMBEDDING_DIM = 32
NUM_IDX = SEQ_LEN * BATCH
NUM_WORKERS = 32
PER_WORKER = NUM_IDX // NUM_WORKERS
STREAM_W = 128
CHUNK = 640
NSTREAM = CHUNK // STREAM_W
NCHUNK = PER_WORKER // CHUNK
NBUF = 2


def _gather_rows(weight, idx_grp):
    mesh = plsc.VectorSubcoreMesh(core_axis_name="c", subcore_axis_name="s")

    @pl.kernel(
        out_type=jax.ShapeDtypeStruct(
            (NUM_WORKERS, NCHUNK, CHUNK, EMBEDDING_DIM), weight.dtype
        ),
        mesh=mesh,
        scratch_types=[
            pltpu.VMEM((NCHUNK, NSTREAM, STREAM_W), jnp.int32),
            pltpu.VMEM((NBUF, CHUNK, EMBEDDING_DIM), jnp.float32),
            pltpu.VMEM_SHARED((16, NBUF, CHUNK, EMBEDDING_DIM), jnp.float32),
        ]
        + [pltpu.SemaphoreType.DMA] * (2 * NBUF),
        compiler_params=pltpu.CompilerParams(use_tc_tiling_on_sc=False),
    )
    def gather_kernel(w_hbm, i_hbm, o_hbm, idx_v, rows_v, sp_rows, *sems):
        sid = lax.axis_index("s")
        wid = sid * 2 + lax.axis_index("c")
        gsems = sems[:NBUF]
        osems = sems[NBUF:]

        pltpu.sync_copy(i_hbm.at[wid], idx_v)

        def load_and_fire(g, b):
            for j in range(NSTREAM):
                pltpu.async_copy(
                    w_hbm.at[idx_v.at[g, j]],
                    rows_v.at[b, pl.ds(j * STREAM_W, STREAM_W)],
                    gsems[b],
                )

        def drain(g, b):
            for j in range(NSTREAM):
                pltpu.make_async_copy(
                    w_hbm.at[idx_v.at[g, j]],
                    rows_v.at[b, pl.ds(j * STREAM_W, STREAM_W)],
                    gsems[b],
                ).wait()

        for b in range(NBUF):
            load_and_fire(b, b)

        def ring_body(p, carry):
            for b in range(NBUF):
                g = NBUF * p + b
                drain(g, b)

                @pl.when(g >= NBUF)
                def _():
                    pltpu.make_async_copy(
                        sp_rows.at[sid, b], o_hbm.at[wid, g - NBUF], osems[b]
                    ).wait()

                pltpu.sync_copy(rows_v.at[b], sp_rows.at[sid, b])
                pltpu.async_copy(sp_rows.at[sid, b], o_hbm.at[wid, g], osems[b])

                @pl.when(g + NBUF < NCHUNK)
                def _():
                    load_and_fire(g + NBUF, b)

            return carry

        lax.fori_loop(0, NCHUNK // NBUF, ring_body, 0)

        for b in range(NBUF):
            pltpu.make_async_copy(
                sp_rows.at[sid, b], o_hbm.at[wid, NCHUNK - NBUF + b], osems[b]
            ).wait()

    return gather_kernel(weight, idx_grp)


def kernel(inputs, weight):
    idx_grp = inputs.reshape(NUM_WORKERS, NCHUNK, NSTREAM, STREAM_W)
    out = _gather_rows(weight, idx_grp)
    return out.reshape(SEQ_LEN, BATCH, EMBEDDING_DIM)


# speedup vs baseline: 1.0207x; 1.0019x over previous
import jax
import jax.numpy as jnp
from jax import lax
from jax.experimental import pallas as pl
from jax.experimental.pallas import tpu as pltpu
from jax.experimental.pallas import tpu_sc as plsc

SEQ_LEN = 200
BATCH = 4096
EMBEDDING_DIM = 32
NUM_IDX = SEQ_LEN * BATCH
NUM_WORKERS = 32
PER_WORKER = NUM_IDX // NUM_WORKERS
STREAM_W = 128
CHUNK = 256
NSTREAM = CHUNK // STREAM_W
NCHUNK = PER_WORKER // CHUNK
NBUF = 4


def _gather_rows(weight, idx_grp):
    mesh = plsc.VectorSubcoreMesh(core_axis_name="c", subcore_axis_name="s")

    @pl.kernel(
        out_type=jax.ShapeDtypeStruct(
            (NUM_WORKERS, NCHUNK, CHUNK, EMBEDDING_DIM), weight.dtype
        ),
        mesh=mesh,
        scratch_types=[
            pltpu.VMEM((NCHUNK, NSTREAM, STREAM_W), jnp.int32),
            pltpu.VMEM((NBUF, CHUNK, EMBEDDING_DIM), jnp.float32),
            pltpu.VMEM_SHARED((16, NBUF, CHUNK, EMBEDDING_DIM), jnp.float32),
        ]
        + [pltpu.SemaphoreType.DMA] * (3 * NBUF),
        compiler_params=pltpu.CompilerParams(use_tc_tiling_on_sc=False),
    )
    def gather_kernel(w_hbm, i_hbm, o_hbm, idx_v, rows_v, sp_rows, *sems):
        sid = lax.axis_index("s")
        wid = sid * 2 + lax.axis_index("c")
        gsems = sems[:NBUF]
        csems = sems[NBUF : 2 * NBUF]
        osems = sems[2 * NBUF :]

        pltpu.sync_copy(i_hbm.at[wid], idx_v)

        def fire(g, b):
            for j in range(NSTREAM):
                pltpu.async_copy(
                    w_hbm.at[idx_v.at[g, j]],
                    rows_v.at[b, pl.ds(j * STREAM_W, STREAM_W)],
                    gsems[b],
                )

        def drain(g, b):
            for j in range(NSTREAM):
                pltpu.make_async_copy(
                    w_hbm.at[idx_v.at[g, j]],
                    rows_v.at[b, pl.ds(j * STREAM_W, STREAM_W)],
                    gsems[b],
                ).wait()

        for b in range(NBUF):
            fire(b, b)

        def ring_body(p, carry):
            for b in range(NBUF):
                g = NBUF * p + b
                bp = (b - 1) % NBUF
                drain(g, b)

                @pl.when(g >= 1)
                def _():
                    pltpu.make_async_copy(
                        rows_v.at[bp], sp_rows.at[sid, bp], csems[bp]
                    ).wait()
                    pltpu.async_copy(
                        sp_rows.at[sid, bp], o_hbm.at[wid, g - 1], osems[bp]
                    )

                    @pl.when(g - 1 + NBUF < NCHUNK)
                    def _():
                        fire(g - 1 + NBUF, bp)

                @pl.when(g >= NBUF)
                def _():
                    pltpu.make_async_copy(
                        sp_rows.at[sid, b], o_hbm.at[wid, g - NBUF], osems[b]
                    ).wait()

                pltpu.async_copy(rows_v.at[b], sp_rows.at[sid, b], csems[b])

            return carry

        lax.fori_loop(0, NCHUNK // NBUF, ring_body, 0)

        bl = (NCHUNK - 1) % NBUF
        pltpu.make_async_copy(rows_v.at[bl], sp_rows.at[sid, bl], csems[bl]).wait()
        pltpu.async_copy(sp_rows.at[sid, bl], o_hbm.at[wid, NCHUNK - 1], osems[bl])
        for b in range(NBUF):
            pltpu.make_async_copy(
                sp_rows.at[sid, b], o_hbm.at[wid, NCHUNK - NBUF + b], osems[b]
            ).wait()

    return gather_kernel(weight, idx_grp)


def kernel(inputs, weight):
    idx_grp = inputs.reshape(NUM_WORKERS, NCHUNK, NSTREAM, STREAM_W)
    out = _gather_rows(weight, idx_grp)
    return out.reshape(SEQ_LEN, BATCH, EMBEDDING_DIM)
